# baseline (speedup 1.0000x reference)
import functools

import jax
import jax.numpy as jnp
from jax import lax
from jax.experimental import pallas as pl
from jax.experimental.pallas import tpu as pltpu
from jax.experimental.pallas import tpu_sc as plsc

D = 64
NW = 32
NBUF = 4


@jax.jit
def _lookup(idx, table):
    N, CH = idx.shape
    n_ch = N // NW
    n_grp = n_ch // NBUF
    assert n_ch % NBUF == 0

    mesh = plsc.VectorSubcoreMesh(core_axis_name="c", subcore_axis_name="s")

    @functools.partial(
        pl.kernel,
        out_type=jax.ShapeDtypeStruct((N, CH, D), jnp.float32),
        mesh=mesh,
        scratch_types=[
            pltpu.VMEM((n_ch, CH), jnp.int32),
            pltpu.VMEM((NBUF, CH, D), jnp.float32),
            pltpu.SemaphoreType.DMA((NBUF,)),
            pltpu.SemaphoreType.DMA((NBUF,)),
        ],
        compiler_params=pltpu.CompilerParams(use_tc_tiling_on_sc=False),
    )
    def k(idx_hbm, table_hbm, out_hbm, idx_v, rows_v, gsem, ssem):
        wid = lax.axis_index("s") * 2 + lax.axis_index("c")
        base = wid * n_ch
        pltpu.sync_copy(idx_hbm.at[pl.ds(base, n_ch)], idx_v)

        def gather(t, b):
            return pltpu.make_async_copy(
                table_hbm.at[idx_v.at[t]], rows_v.at[b], gsem.at[b])

        def store(t, b):
            return pltpu.make_async_copy(
                rows_v.at[b], out_hbm.at[base + t], ssem.at[b])

        for b in range(NBUF):
            gather(b, b).start()

        def group(g, carry):
            for b in range(NBUF):
                t = g * NBUF + b
                gather(t, b).wait()
                store(t, b).start()

                @pl.when(g + 1 < n_grp)
                def _():
                    store(t, b).wait()
                    gather(t + NBUF, b).start()
            return carry

        lax.fori_loop(0, n_grp, group, 0)

        for b in range(NBUF):
            t = (n_grp - 1) * NBUF + b
            store(t, b).wait()

    return k(idx, table)


def kernel(inputs, emb_edges):
    return _lookup(inputs, emb_edges)


# speedup vs baseline: 1.1545x; 1.1545x over previous
import functools

import jax
import jax.numpy as jnp
from jax import lax
from jax.experimental import pallas as pl
from jax.experimental.pallas import tpu as pltpu
from jax.experimental.pallas import tpu_sc as plsc

D = 64
DP = 128
NW = 32
NBUF = 4


@jax.jit
def _lookup(idx1d, tpad):
    B = idx1d.shape[0]
    b_per_w = B // NW
    n_b = 1024 // NW
    CH = 200
    n_grp = n_b // NBUF
    assert b_per_w == n_b * CH and n_b % NBUF == 0

    mesh = plsc.VectorSubcoreMesh(core_axis_name="c", subcore_axis_name="s")

    @functools.partial(
        pl.kernel,
        out_type=jax.ShapeDtypeStruct((1024, 200, DP), jnp.float32),
        mesh=mesh,
        scratch_types=[
            pltpu.VMEM((b_per_w,), jnp.int32),
            pltpu.VMEM((NBUF, CH, DP), jnp.float32),
            pltpu.SemaphoreType.DMA((NBUF,)),
            pltpu.SemaphoreType.DMA((NBUF,)),
        ],
    )
    def k(idx_hbm, table_hbm, out_hbm, idx_v, rows_v, gsem, ssem):
        wid = lax.axis_index("s") * 2 + lax.axis_index("c")
        base_b = wid * n_b
        pltpu.sync_copy(idx_hbm.at[pl.ds(wid * b_per_w, b_per_w)], idx_v)

        def gather(t, b):
            return pltpu.make_async_copy(
                table_hbm.at[idx_v.at[pl.ds(t * CH, CH)]], rows_v.at[b],
                gsem.at[b])

        def store(t, b):
            return pltpu.make_async_copy(
                rows_v.at[b], out_hbm.at[base_b + t], ssem.at[b])

        for b in range(NBUF):
            gather(b, b).start()

        def group(g, carry):
            for b in range(NBUF):
                t = g * NBUF + b
                gather(t, b).wait()
                store(t, b).start()

                @pl.when(g + 1 < n_grp)
                def _():
                    store(t, b).wait()
                    gather(t + NBUF, b).start()
            return carry

        lax.fori_loop(0, n_grp, group, 0)

        for b in range(NBUF):
            t = (n_grp - 1) * NBUF + b
            store(t, b).wait()

    return k(idx1d, tpad)


def kernel(inputs, emb_edges):
    idx1d = inputs.reshape(-1)
    tpad = jnp.pad(emb_edges, ((0, 0), (0, DP - D)))
    outp = _lookup(idx1d, tpad)
    return outp[:, :, :D]
